# baseline (speedup 1.0000x reference)
import functools

import jax
import jax.numpy as jnp
from jax import lax
from jax.experimental import pallas as pl
from jax.experimental.pallas import tpu as pltpu
from jax.experimental.pallas import tpu_sc as plsc

B = 4
S = 8192
D = 1024

S_SPLIT = 5632
SREM = S - S_SPLIT
NW = 32
SPW = SREM // NW
CH = 8
NCHUNK = SPW // CH
UNROLL = 8

_BS_TC = 512
_BS_CP = 512

_mesh = plsc.VectorSubcoreMesh(core_axis_name="c", subcore_axis_name="s")


@functools.partial(
    pl.kernel,
    mesh=_mesh,
    out_type=jax.ShapeDtypeStruct((B * SREM, D), jnp.float32),
    scratch_types=[pltpu.VMEM((CH, D), jnp.float32)] * 10
    + [pltpu.SemaphoreType.DMA] * 4,
)
def _sc_add(x_hbm, pos_hbm, out_hbm,
            pA, xA0, xA1, xA2, xA3, pB, xB0, xB1, xB2, xB3,
            siA, siB, soA, soB):
    wid = lax.axis_index("s") * 2 + lax.axis_index("c")
    s0w = wid * SPW
    GA = (pA, (xA0, xA1, xA2, xA3), siA, soA)
    GB = (pB, (xB0, xB1, xB2, xB3), siB, soB)

    def ins(g, c):
        s0 = s0w + c * CH
        ds = [pltpu.make_async_copy(pos_hbm.at[pl.ds(S_SPLIT + s0, CH)], g[0], g[2])]
        ds += [
            pltpu.make_async_copy(
                x_hbm.at[pl.ds(b * S + S_SPLIT + s0, CH)], g[1][b], g[2]
            )
            for b in range(B)
        ]
        return ds

    def outs(g, c):
        s0 = s0w + c * CH
        return [
            pltpu.make_async_copy(g[1][b], out_hbm.at[pl.ds(b * SREM + s0, CH)], g[3])
            for b in range(B)
        ]

    def fire(ds):
        for d in ds:
            d.start()

    def drain(ds):
        for d in ds:
            d.wait()

    def compute(g):
        p, xs = g[0], g[1]
        for r in range(CH):
            def body(i, carry):
                for u in range(UNROLL):
                    sl = pl.ds(i * 16 * UNROLL + u * 16, 16)
                    pv = p[r, sl]
                    for b in range(B):
                        xs[b][r, sl] = xs[b][r, sl] + pv
                return carry
            lax.fori_loop(0, D // 16 // UNROLL, body, 0)

    fire(ins(GA, 0))
    fire(ins(GB, 1))
    drain(ins(GA, 0))
    compute(GA)
    fire(outs(GA, 0))

    def loop_body(i, carry):
        c0 = 1 + 2 * i
        drain(outs(GA, c0 - 1))
        fire(ins(GA, c0 + 1))
        drain(ins(GB, c0))
        compute(GB)
        fire(outs(GB, c0))

        drain(outs(GB, c0))
        fire(ins(GB, c0 + 2))
        drain(ins(GA, c0 + 1))
        compute(GA)
        fire(outs(GA, c0 + 1))
        return carry

    lax.fori_loop(0, (NCHUNK - 2) // 2, loop_body, 0)

    c_last = NCHUNK - 1
    drain(outs(GA, c_last - 1))
    drain(ins(GB, c_last))
    compute(GB)
    fire(outs(GB, c_last))
    drain(outs(GB, c_last))


def _tc_add_kernel(x_ref, pos_ref, out_ref):
    out_ref[...] = x_ref[...] + pos_ref[...][None, :, :]


def _tc_main(x, pos_table):
    return pl.pallas_call(
        _tc_add_kernel,
        grid=(S_SPLIT // _BS_TC, B),
        in_specs=[
            pl.BlockSpec((1, _BS_TC, D), lambda s, b: (b, s, 0)),
            pl.BlockSpec((_BS_TC, D), lambda s, b: (s, 0)),
        ],
        out_specs=pl.BlockSpec((1, _BS_TC, D), lambda s, b: (b, s, 0)),
        out_shape=jax.ShapeDtypeStruct((B, S, D), jnp.float32),
        compiler_params=pltpu.CompilerParams(
            dimension_semantics=("arbitrary", "arbitrary"),
        ),
    )(x, pos_table)


def _copy_kernel(full_ref, sc_ref, out_ref):
    out_ref[...] = sc_ref[...]


def _stitch(full, sc_part):
    return pl.pallas_call(
        _copy_kernel,
        grid=(SREM // _BS_CP, B),
        in_specs=[
            pl.BlockSpec(memory_space=pl.ANY),
            pl.BlockSpec((1, _BS_CP, D), lambda s, b: (b, s, 0)),
        ],
        out_specs=pl.BlockSpec(
            (1, _BS_CP, D), lambda s, b: (b, s + S_SPLIT // _BS_CP, 0)
        ),
        out_shape=jax.ShapeDtypeStruct((B, S, D), jnp.float32),
        input_output_aliases={0: 0},
        compiler_params=pltpu.CompilerParams(
            dimension_semantics=("arbitrary", "arbitrary"),
        ),
    )(full, sc_part)


def kernel(x, pos_table):
    sc_out = _sc_add(x.reshape(B * S, D), pos_table)
    tc_out = _tc_main(x, pos_table)
    return _stitch(tc_out, sc_out.reshape(B, SREM, D))


# speedup vs baseline: 1.0005x; 1.0005x over previous
import functools

import jax
import jax.numpy as jnp
from jax import lax
from jax.experimental import pallas as pl
from jax.experimental.pallas import tpu as pltpu
from jax.experimental.pallas import tpu_sc as plsc

B = 4
S = 8192
D = 1024

S_SPLIT = 5632
SREM = S - S_SPLIT
NW = 32
SPW = SREM // NW
CH = 8
NCHUNK = SPW // CH
UNROLL = 8

_BS_TC = 512
_BS_CP = 512

_mesh = plsc.VectorSubcoreMesh(core_axis_name="c", subcore_axis_name="s")


@functools.partial(
    pl.kernel,
    mesh=_mesh,
    out_type=jax.ShapeDtypeStruct((B * SREM, D), jnp.float32),
    scratch_types=[pltpu.VMEM((CH, D), jnp.float32)] * 10
    + [pltpu.SemaphoreType.DMA] * 4,
)
def _sc_add(x_hbm, pos_hbm, out_hbm,
            pA, xA0, xA1, xA2, xA3, pB, xB0, xB1, xB2, xB3,
            siA, siB, soA, soB):
    wid = lax.axis_index("s") * 2 + lax.axis_index("c")
    s0w = wid * SPW
    GA = (pA, (xA0, xA1, xA2, xA3), siA, soA)
    GB = (pB, (xB0, xB1, xB2, xB3), siB, soB)

    def ins(g, c):
        s0 = s0w + c * CH
        ds = [pltpu.make_async_copy(pos_hbm.at[pl.ds(S_SPLIT + s0, CH)], g[0], g[2])]
        ds += [
            pltpu.make_async_copy(
                x_hbm.at[pl.ds(b * S + S_SPLIT + s0, CH)], g[1][b], g[2]
            )
            for b in range(B)
        ]
        return ds

    def outs(g, c):
        s0 = s0w + c * CH
        return [
            pltpu.make_async_copy(g[1][b], out_hbm.at[pl.ds(b * SREM + s0, CH)], g[3])
            for b in range(B)
        ]

    def fire(ds):
        for d in ds:
            d.start()

    def drain(ds):
        for d in ds:
            d.wait()

    def compute(g):
        p, xs = g[0], g[1]
        for r in range(CH):
            def body(i, carry):
                for u in range(UNROLL):
                    sl = pl.ds(i * 16 * UNROLL + u * 16, 16)
                    pv = p[r, sl]
                    for b in range(B):
                        xs[b][r, sl] = xs[b][r, sl] + pv
                return carry
            lax.fori_loop(0, D // 16 // UNROLL, body, 0)

    fire(ins(GA, 0))
    fire(ins(GB, 1))
    drain(ins(GA, 0))
    compute(GA)
    fire(outs(GA, 0))

    def loop_body(i, carry):
        c0 = 1 + 2 * i
        drain(outs(GA, c0 - 1))
        fire(ins(GA, c0 + 1))
        drain(ins(GB, c0))
        compute(GB)
        fire(outs(GB, c0))

        drain(outs(GB, c0))
        fire(ins(GB, c0 + 2))
        drain(ins(GA, c0 + 1))
        compute(GA)
        fire(outs(GA, c0 + 1))
        return carry

    lax.fori_loop(0, (NCHUNK - 2) // 2, loop_body, 0)

    c_last = NCHUNK - 1
    drain(outs(GA, c_last - 1))
    drain(ins(GB, c_last))
    compute(GB)
    fire(outs(GB, c_last))
    drain(outs(GB, c_last))


def _tc_add_kernel(x_ref, pos_ref, out_ref):
    out_ref[...] = x_ref[...] + pos_ref[...][None, :, :]


def _tc_main(x, pos_table):
    return pl.pallas_call(
        _tc_add_kernel,
        grid=(S_SPLIT // _BS_TC, B),
        in_specs=[
            pl.BlockSpec((1, _BS_TC, D), lambda s, b: (b, s, 0)),
            pl.BlockSpec((_BS_TC, D), lambda s, b: (s, 0)),
        ],
        out_specs=pl.BlockSpec((1, _BS_TC, D), lambda s, b: (b, s, 0)),
        out_shape=jax.ShapeDtypeStruct((B, S, D), jnp.float32),
        compiler_params=pltpu.CompilerParams(
            dimension_semantics=("arbitrary", "arbitrary"),
        ),
    )(x, pos_table)


def _copy_kernel(full_ref, sc_ref, out_ref):
    out_ref[...] = sc_ref[...]


def _stitch(full, sc_part):
    return pl.pallas_call(
        _copy_kernel,
        grid=(SREM // _BS_CP, B),
        in_specs=[
            pl.BlockSpec(memory_space=pl.ANY),
            pl.BlockSpec((1, _BS_CP, D), lambda s, b: (b, s, 0)),
        ],
        out_specs=pl.BlockSpec(
            (1, _BS_CP, D), lambda s, b: (b, s + S_SPLIT // _BS_CP, 0)
        ),
        out_shape=jax.ShapeDtypeStruct((B, S, D), jnp.float32),
        input_output_aliases={0: 0},
        compiler_params=pltpu.CompilerParams(
            dimension_semantics=("arbitrary", "arbitrary"),
        ),
    )(full, sc_part)


def kernel(x, pos_table):
    tc_out = _tc_main(x, pos_table)
    sc_out = _sc_add(x.reshape(B * S, D), pos_table)
    return _stitch(tc_out, sc_out.reshape(B, SREM, D))


# speedup vs baseline: 1.1037x; 1.1032x over previous
import functools

import jax
import jax.numpy as jnp
from jax import lax
from jax.experimental import pallas as pl
from jax.experimental.pallas import tpu as pltpu
from jax.experimental.pallas import tpu_sc as plsc

B = 4
S = 8192
D = 1024
NW = 32
SPW = S // NW
CH = 8
NCHUNK = SPW // CH
UNROLL = 8

_mesh = plsc.VectorSubcoreMesh(core_axis_name="c", subcore_axis_name="s")


@functools.partial(
    pl.kernel,
    mesh=_mesh,
    out_type=jax.ShapeDtypeStruct((B * S, D), jnp.float32),
    scratch_types=[pltpu.VMEM((CH, D), jnp.float32)] * 15
    + [pltpu.SemaphoreType.DMA] * 6,
)
def _sc_add(x_hbm, pos_hbm, out_hbm,
            p0, x00, x01, x02, x03,
            p1, x10, x11, x12, x13,
            p2, x20, x21, x22, x23,
            si0, si1, si2, so0, so1, so2):
    wid = lax.axis_index("s") * 2 + lax.axis_index("c")
    s0w = wid * SPW
    G = (
        (p0, (x00, x01, x02, x03), si0, so0),
        (p1, (x10, x11, x12, x13), si1, so1),
        (p2, (x20, x21, x22, x23), si2, so2),
    )

    def ins(g, c):
        s0 = s0w + c * CH
        ds = [pltpu.make_async_copy(pos_hbm.at[pl.ds(s0, CH)], g[0], g[2])]
        ds += [
            pltpu.make_async_copy(x_hbm.at[pl.ds(b * S + s0, CH)], g[1][b], g[2])
            for b in range(B)
        ]
        return ds

    def outs(g, c):
        s0 = s0w + c * CH
        return [
            pltpu.make_async_copy(g[1][b], out_hbm.at[pl.ds(b * S + s0, CH)], g[3])
            for b in range(B)
        ]

    def fire(ds):
        for d in ds:
            d.start()

    def drain(ds):
        for d in ds:
            d.wait()

    def compute(g):
        p, xs = g[0], g[1]
        for r in range(CH):
            def body(i, carry):
                for u in range(UNROLL):
                    sl = pl.ds(i * 16 * UNROLL + u * 16, 16)
                    pv = p[r, sl]
                    for b in range(B):
                        xs[b][r, sl] = xs[b][r, sl] + pv
                return carry
            lax.fori_loop(0, D // 16 // UNROLL, body, 0)

    def step(c, k, ok, fire_c=None, drain_c=None):
        g = G[k]
        drain(ins(g, c))
        compute(g)
        fire(outs(g, c))
        if drain_c is not None:
            drain(outs(G[ok], drain_c))
        if fire_c is not None:
            fire(ins(G[ok], fire_c))

    fire(ins(G[0], 0))
    fire(ins(G[1], 1))
    step(0, 0, 2, fire_c=2)
    step(1, 1, 0, fire_c=3, drain_c=0)

    def loop_body(i, carry):
        c0 = 2 + 3 * i
        step(c0, 2, 1, fire_c=c0 + 2, drain_c=c0 - 1)
        step(c0 + 1, 0, 2, fire_c=c0 + 3, drain_c=c0)
        step(c0 + 2, 1, 0, fire_c=c0 + 4, drain_c=c0 + 1)
        return carry

    lax.fori_loop(0, (NCHUNK - 5) // 3, loop_body, 0)

    step(NCHUNK - 3, 2, 1, fire_c=NCHUNK - 1, drain_c=NCHUNK - 4)
    step(NCHUNK - 2, 0, 2, drain_c=NCHUNK - 3)
    step(NCHUNK - 1, 1, 0, drain_c=NCHUNK - 2)
    drain(outs(G[(NCHUNK - 1) % 3], NCHUNK - 1))


def kernel(x, pos_table):
    out = _sc_add(x.reshape(B * S, D), pos_table)
    return out.reshape(B, S, D)
